# baseline (speedup 1.0000x reference)
import functools

import jax
import jax.numpy as jnp
from jax import lax
from jax.experimental import pallas as pl
from jax.experimental.pallas import tpu as pltpu
from jax.experimental.pallas import tpu_sc as plsc

VOCAB = 1000000
SEQ_LEN = 200
DIM = 64
BATCH = 4096

NUM_CORES = 2
NUM_SUBCORES = 16
NUM_WORKERS = NUM_CORES * NUM_SUBCORES
BCOLS = BATCH // NUM_WORKERS
POS_PAD = 256
NBUF = 4
NGROUPS = SEQ_LEN // NBUF
LANES = 16
BVREGS = BCOLS // LANES


@jax.jit
def _sc_embed(inputs_t, token_table, pos_t):
  mesh = plsc.VectorSubcoreMesh(
      core_axis_name="c", subcore_axis_name="s",
      num_cores=NUM_CORES, num_subcores=NUM_SUBCORES)

  scratch = (
      [pltpu.VMEM((DIM, POS_PAD), jnp.float32)]
      + [pltpu.VMEM((BCOLS,), jnp.int32) for _ in range(NBUF)]
      + [pltpu.VMEM((BCOLS, DIM), jnp.float32) for _ in range(NBUF)]
      + [pltpu.VMEM((DIM, BCOLS), jnp.float32) for _ in range(NBUF)]
      + [pltpu.SemaphoreType.DMA for _ in range(3 * NBUF)]
  )

  @functools.partial(
      pl.kernel,
      out_type=jax.ShapeDtypeStruct((SEQ_LEN, DIM, BATCH), jnp.float32),
      mesh=mesh,
      scratch_types=scratch,
      compiler_params=pltpu.CompilerParams(use_tc_tiling_on_sc=False,
                                           needs_layout_passes=False),
  )
  def k(inputs_hbm, table_hbm, pos_hbm, out_hbm, pos_v, *bufs):
    idx = bufs[:NBUF]
    acc = bufs[NBUF:2 * NBUF]
    accT = bufs[2 * NBUF:3 * NBUF]
    si = bufs[3 * NBUF:4 * NBUF]
    sg = bufs[4 * NBUF:5 * NBUF]
    sw = bufs[5 * NBUF:6 * NBUF]

    wid = lax.axis_index("s") * NUM_CORES + lax.axis_index("c")
    b0 = wid * BCOLS

    pltpu.sync_copy(pos_hbm, pos_v)
    lane_iota = lax.iota(jnp.int32, LANES)

    def start_idx(l, b):
      pltpu.async_copy(inputs_hbm.at[l, pl.ds(b0, BCOLS)], idx[b], si[b])

    def wait_idx(b):
      pltpu.make_async_copy(
          inputs_hbm.at[0, pl.ds(0, BCOLS)], idx[b], si[b]).wait()

    def start_gather(b):
      return pltpu.async_copy(table_hbm.at[idx[b]], acc[b], sg[b])

    def transpose_add(l, b):
      a = acc[b]
      t = accT[b]

      lvec = jnp.full((LANES,), l, jnp.int32)

      @pl.loop(0, DIM)
      def _(d):
        dvec = jnp.full((LANES,), d, jnp.int32)
        p = plsc.load_gather(pos_v, [dvec, lvec])
        for j in range(BVREGS):
          bvec = lane_iota + (j * LANES)
          vals = plsc.load_gather(a, [bvec, dvec])
          t[d, pl.ds(j * LANES, LANES)] = vals + p

    def start_write(l, b):
      pltpu.async_copy(accT[b], out_hbm.at[l, :, pl.ds(b0, BCOLS)], sw[b])

    def wait_write(b):
      pltpu.make_async_copy(
          accT[b], out_hbm.at[0, :, pl.ds(0, BCOLS)], sw[b]).wait()

    for b in range(NBUF):
      start_idx(b, b)

    handles = []
    for b in range(NBUF):
      wait_idx(b)
      handles.append(start_gather(b))
    for b in range(NBUF):
      handles[b].wait()
      transpose_add(b, b)
      start_write(b, b)
      start_idx(NBUF + b, b)

    @pl.loop(1, NGROUPS)
    def _(o):
      l0 = o * NBUF
      hs = []
      for b in range(NBUF):
        wait_idx(b)
        wait_write(b)
        hs.append(start_gather(b))
      for b in range(NBUF):
        hs[b].wait()
        transpose_add(l0 + b, b)
        start_write(l0 + b, b)

        @pl.when(l0 + b + NBUF < SEQ_LEN)
        def _():
          start_idx(l0 + b + NBUF, b)

    for b in range(NBUF):
      wait_write(b)

  return k(inputs_t, token_table, pos_t)


def kernel(inputs, token_table, position_table):
  inputs_t = inputs.astype(jnp.int32).T
  pos_t = jnp.pad(position_table.T, ((0, 0), (0, POS_PAD - SEQ_LEN)))
  out_t = _sc_embed(inputs_t, token_table, pos_t)
  return jnp.transpose(out_t, (2, 0, 1))


# speedup vs baseline: 1.8636x; 1.8636x over previous
import functools

import jax
import jax.numpy as jnp
from jax import lax
from jax.experimental import pallas as pl
from jax.experimental.pallas import tpu as pltpu
from jax.experimental.pallas import tpu_sc as plsc

VOCAB = 1000000
SEQ_LEN = 200
DIM = 64
BATCH = 4096

NUM_CORES = 2
NUM_SUBCORES = 16
NUM_WORKERS = NUM_CORES * NUM_SUBCORES
BCOLS = BATCH // NUM_WORKERS
DT = DIM // 8
BPAD = BCOLS + 1
NBUF = 4
NGROUPS = SEQ_LEN // NBUF
LANES = 16
KV = DIM // LANES


@jax.jit
def _sc_embed(inputs_t, token_table, position_table):
  mesh = plsc.VectorSubcoreMesh(
      core_axis_name="c", subcore_axis_name="s",
      num_cores=NUM_CORES, num_subcores=NUM_SUBCORES)

  scratch = (
      [pltpu.VMEM((SEQ_LEN, DIM), jnp.float32)]
      + [pltpu.VMEM((BCOLS,), jnp.int32) for _ in range(NBUF)]
      + [pltpu.VMEM((BCOLS, DIM), jnp.float32) for _ in range(NBUF)]
      + [pltpu.VMEM((DT, 8, BPAD), jnp.float32) for _ in range(NBUF)]
      + [pltpu.SemaphoreType.DMA for _ in range(3 * NBUF)]
  )

  @functools.partial(
      pl.kernel,
      out_type=jax.ShapeDtypeStruct((SEQ_LEN, DT, NUM_WORKERS, 8, BCOLS),
                                    jnp.float32),
      mesh=mesh,
      scratch_types=scratch,
      compiler_params=pltpu.CompilerParams(use_tc_tiling_on_sc=False,
                                           needs_layout_passes=False),
  )
  def k(inputs_hbm, table_hbm, pos_hbm, out_hbm, pos_v, *bufs):
    idx = bufs[:NBUF]
    acc = bufs[NBUF:2 * NBUF]
    accT = bufs[2 * NBUF:3 * NBUF]
    si = bufs[3 * NBUF:4 * NBUF]
    sg = bufs[4 * NBUF:5 * NBUF]
    sw = bufs[5 * NBUF:6 * NBUF]

    wid = lax.axis_index("s") * NUM_CORES + lax.axis_index("c")
    b0 = wid * BCOLS

    pltpu.sync_copy(pos_hbm, pos_v)
    lane_iota = lax.iota(jnp.int32, LANES)
    ti_vecs = [(lane_iota + 16 * kk) >> 3 for kk in range(KV)]
    r_vecs = [(lane_iota + 16 * kk) & 7 for kk in range(KV)]

    def start_idx(l, b):
      pltpu.async_copy(inputs_hbm.at[l, pl.ds(b0, BCOLS)], idx[b], si[b])

    def wait_idx(b):
      pltpu.make_async_copy(
          inputs_hbm.at[0, pl.ds(0, BCOLS)], idx[b], si[b]).wait()

    def start_gather(b):
      return pltpu.async_copy(table_hbm.at[idx[b]], acc[b], sg[b])

    def transpose_add(l, b):
      a = acc[b]
      t = accT[b]
      pv = [pos_v[l, pl.ds(16 * kk, LANES)] for kk in range(KV)]

      @pl.loop(0, BCOLS, unroll=2)
      def _(bb):
        bvec = jnp.full((LANES,), bb, jnp.int32)
        for kk in range(KV):
          vals = a[bb, pl.ds(16 * kk, LANES)] + pv[kk]
          plsc.store_scatter(t, [ti_vecs[kk], r_vecs[kk], bvec], vals)

    def start_writes(l, b):
      for ti in range(DT):
        pltpu.async_copy(
            accT[b].at[ti, :, pl.ds(0, BCOLS)],
            out_hbm.at[l, ti, wid], sw[b])

    def wait_writes(b):
      for ti in range(DT):
        pltpu.make_async_copy(
            accT[b].at[ti, :, pl.ds(0, BCOLS)],
            out_hbm.at[0, ti, 0], sw[b]).wait()

    for b in range(NBUF):
      start_idx(b, b)

    handles = []
    for b in range(NBUF):
      wait_idx(b)
      handles.append(start_gather(b))
    for b in range(NBUF):
      handles[b].wait()
      transpose_add(b, b)
      start_writes(b, b)
      start_idx(NBUF + b, b)

    @pl.loop(1, NGROUPS)
    def _(o):
      l0 = o * NBUF
      hs = []
      for b in range(NBUF):
        wait_idx(b)
        wait_writes(b)
        hs.append(start_gather(b))
      for b in range(NBUF):
        hs[b].wait()
        transpose_add(l0 + b, b)
        start_writes(l0 + b, b)

        @pl.when(l0 + b + NBUF < SEQ_LEN)
        def _():
          start_idx(l0 + b + NBUF, b)

    for b in range(NBUF):
      wait_writes(b)

  return k(inputs_t, token_table, position_table)


def kernel(inputs, token_table, position_table):
  inputs_t = inputs.astype(jnp.int32).T
  out6 = _sc_embed(inputs_t, token_table, position_table)
  out = jnp.transpose(out6, (2, 4, 0, 1, 3))
  return out.reshape(BATCH, SEQ_LEN, DIM)


# speedup vs baseline: 1.8793x; 1.0085x over previous
import functools

import jax
import jax.numpy as jnp
from jax import lax
from jax.experimental import pallas as pl
from jax.experimental.pallas import tpu as pltpu
from jax.experimental.pallas import tpu_sc as plsc

VOCAB = 1000000
SEQ_LEN = 200
DIM = 64
BATCH = 4096

NUM_CORES = 2
NUM_SUBCORES = 16
NUM_WORKERS = NUM_CORES * NUM_SUBCORES
BCOLS = BATCH // NUM_WORKERS
DT = DIM // 8
BPAD = BCOLS + 1
NBUF = 4
NGROUPS = SEQ_LEN // NBUF
LANES = 16
KV = DIM // LANES


@jax.jit
def _sc_embed(inputs_t, token_table, position_table):
  mesh = plsc.VectorSubcoreMesh(
      core_axis_name="c", subcore_axis_name="s",
      num_cores=NUM_CORES, num_subcores=NUM_SUBCORES)

  scratch = (
      [pltpu.VMEM((SEQ_LEN, DIM), jnp.float32)]
      + [pltpu.VMEM((BCOLS,), jnp.int32) for _ in range(NBUF)]
      + [pltpu.VMEM((BCOLS, DIM), jnp.float32) for _ in range(NBUF)]
      + [pltpu.VMEM((DIM, BPAD), jnp.float32) for _ in range(NBUF)]
      + [pltpu.SemaphoreType.DMA for _ in range(3 * NBUF)]
  )

  @functools.partial(
      pl.kernel,
      out_type=jax.ShapeDtypeStruct((SEQ_LEN, DT, NUM_WORKERS, 8, BCOLS),
                                    jnp.float32),
      mesh=mesh,
      scratch_types=scratch,
      compiler_params=pltpu.CompilerParams(use_tc_tiling_on_sc=False,
                                           needs_layout_passes=False),
  )
  def k(inputs_hbm, table_hbm, pos_hbm, out_hbm, pos_v, *bufs):
    idx = bufs[:NBUF]
    acc = bufs[NBUF:2 * NBUF]
    accT = bufs[2 * NBUF:3 * NBUF]
    si = bufs[3 * NBUF:4 * NBUF]
    sg = bufs[4 * NBUF:5 * NBUF]
    sw = bufs[5 * NBUF:6 * NBUF]

    wid = lax.axis_index("s") * NUM_CORES + lax.axis_index("c")
    b0 = wid * BCOLS

    pltpu.sync_copy(pos_hbm, pos_v)
    lane_iota = lax.iota(jnp.int32, LANES)
    d_rows = [lane_iota + 16 * kk for kk in range(KV)]

    def start_idx(l, b):
      pltpu.async_copy(inputs_hbm.at[l, pl.ds(b0, BCOLS)], idx[b], si[b])

    def wait_idx(b):
      pltpu.make_async_copy(
          inputs_hbm.at[0, pl.ds(0, BCOLS)], idx[b], si[b]).wait()

    def start_gather(b):
      return pltpu.async_copy(table_hbm.at[idx[b]], acc[b], sg[b])

    def transpose_add(l, b):
      a = acc[b]
      t = accT[b]
      pv = [pos_v[l, pl.ds(16 * kk, LANES)] for kk in range(KV)]

      @pl.loop(0, BCOLS, unroll=4)
      def _(bb):
        bvec = jnp.full((LANES,), bb, jnp.int32)
        for kk in range(KV):
          vals = a[bb, pl.ds(16 * kk, LANES)] + pv[kk]
          plsc.store_scatter(t, [d_rows[kk], bvec], vals)

    def start_writes(l, b):
      for ti in range(DT):
        for r in range(8):
          pltpu.async_copy(
              accT[b].at[ti * 8 + r, pl.ds(0, BCOLS)],
              out_hbm.at[l, ti, wid, r], sw[b])

    def wait_writes(b):
      for ti in range(DT):
        for r in range(8):
          pltpu.make_async_copy(
              accT[b].at[ti * 8 + r, pl.ds(0, BCOLS)],
              out_hbm.at[0, ti, 0, r], sw[b]).wait()

    for b in range(NBUF):
      start_idx(b, b)

    handles = []
    for b in range(NBUF):
      wait_idx(b)
      handles.append(start_gather(b))
    for b in range(NBUF):
      handles[b].wait()
      transpose_add(b, b)
      start_writes(b, b)
      start_idx(NBUF + b, b)

    @pl.loop(1, NGROUPS)
    def _(o):
      l0 = o * NBUF
      hs = []
      for b in range(NBUF):
        wait_idx(b)
        wait_writes(b)
        hs.append(start_gather(b))
      for b in range(NBUF):
        hs[b].wait()
        transpose_add(l0 + b, b)
        start_writes(l0 + b, b)

        @pl.when(l0 + b + NBUF < SEQ_LEN)
        def _():
          start_idx(l0 + b + NBUF, b)

    for b in range(NBUF):
      wait_writes(b)

  return k(inputs_t, token_table, position_table)


def kernel(inputs, token_table, position_table):
  inputs_t = inputs.astype(jnp.int32).T
  out6 = _sc_embed(inputs_t, token_table, position_table)
  out = jnp.transpose(out6, (2, 4, 0, 1, 3))
  return out.reshape(BATCH, SEQ_LEN, DIM)
